# baseline (speedup 1.0000x reference)
import jax
import jax.numpy as jnp
from jax.experimental import pallas as pl
from jax.experimental.pallas import tpu as pltpu

N3 = 300000
N = N3 // 3
D = 128
B = 4000
NZ = N // B


def _log_sigmoid(x):
    return jnp.minimum(x, 0.0) - jnp.log1p(jnp.exp(-jnp.abs(x)))


def _body(z_ref, zp_ref, zn_ref, wz_ref, wp_ref, wn_ref, o_ref, s_ref, sc_ref):
    g = pl.program_id(0)

    @pl.when(g == 0)
    def _init():
        s_ref[...] = jnp.zeros_like(s_ref)
        sc_ref[0] = 0.0
        sc_ref[1] = 0.0
        sc_ref[2] = 0.0
        sc_ref[3] = 0.0

    z = z_ref[...]
    zp = zp_ref[...]
    zn = zn_ref[...]
    wz = wz_ref[0, 0, :]
    wp = wp_ref[0, 0, :]
    wn = wn_ref[0, 0, :]

    @pl.when(g < NZ)
    def _phase_a():
        dp = jnp.sum(z * zp, axis=1)
        dn = jnp.sum(z * zn, axis=1)
        sc_ref[0] += jnp.sum(_log_sigmoid(dp))
        sc_ref[1] += jnp.sum(_log_sigmoid(-dn))
        s_ref[0, :] += (jnp.sum(z * wz[:, None], axis=0)
                        + jnp.sum(zp * wp[:, None], axis=0)
                        + jnp.sum(zn * wn[:, None], axis=0))
        sc_ref[2] += jnp.sum(wz) + jnp.sum(wp) + jnp.sum(wn)

    @pl.when(g >= NZ)
    def _phase_b():
        mu = s_ref[0, :] / jnp.maximum(sc_ref[2], 1.0)

        def contrib(x, w):
            d = x - mu[None, :]
            return jnp.sum(jnp.sqrt(jnp.sum(d * d, axis=1)) * w)

        sc_ref[3] += contrib(z, wz) + contrib(zp, wp) + contrib(zn, wn)

    @pl.when(g == 2 * NZ - 1)
    def _fin():
        coag = sc_ref[3]
        sig = 1.0 / (1.0 + jnp.exp(-coag))
        total = -(sc_ref[0] + sc_ref[1]) / N + sig - 0.5
        o_ref[...] = jnp.full((1, 1), total, dtype=jnp.float32)


def kernel(out, mask):
    w = mask.astype(jnp.float32).reshape(3 * NZ, 1, B)

    def omap(t):
        return lambda g: (t * NZ + g % NZ, 0)

    def wmap(t):
        return lambda g: (t * NZ + g % NZ, 0, 0)

    res = pl.pallas_call(
        _body,
        grid=(2 * NZ,),
        in_specs=[
            pl.BlockSpec((B, D), omap(0)),
            pl.BlockSpec((B, D), omap(1)),
            pl.BlockSpec((B, D), omap(2)),
            pl.BlockSpec((1, 1, B), wmap(0)),
            pl.BlockSpec((1, 1, B), wmap(1)),
            pl.BlockSpec((1, 1, B), wmap(2)),
        ],
        out_specs=pl.BlockSpec((1, 1), lambda g: (0, 0)),
        out_shape=jax.ShapeDtypeStruct((1, 1), jnp.float32),
        scratch_shapes=[
            pltpu.VMEM((1, D), jnp.float32),
            pltpu.SMEM((4,), jnp.float32),
        ],
        compiler_params=pltpu.CompilerParams(
            dimension_semantics=("arbitrary",),
        ),
    )(out, out, out, w, w, w)
    return res[0, 0]
